# baseline (speedup 1.0000x reference)
import jax
import jax.numpy as jnp
from jax.experimental import pallas as pl
from jax.experimental.pallas import tpu as pltpu

_ROWS = 2048
_COLS = 5023 * 3
_TOTAL = _ROWS * _COLS
_BLOCK_ROWS = 32


def _mse_block_kernel(p_ref, r_ref, out_ref):
    i = pl.program_id(0)
    d = p_ref[...] - r_ref[...]
    s = jnp.sum(d * d)

    @pl.when(i == 0)
    def _init():
        out_ref[0] = 0.0

    out_ref[0] += s


def kernel(pred_vertices, ref_vertices):
    p = pred_vertices.reshape(_ROWS, _COLS)
    r = ref_vertices.reshape(_ROWS, _COLS)
    grid = (_ROWS // _BLOCK_ROWS,)
    total = pl.pallas_call(
        _mse_block_kernel,
        grid=grid,
        in_specs=[
            pl.BlockSpec((_BLOCK_ROWS, _COLS), lambda i: (i, 0)),
            pl.BlockSpec((_BLOCK_ROWS, _COLS), lambda i: (i, 0)),
        ],
        out_specs=pl.BlockSpec(memory_space=pltpu.MemorySpace.SMEM),
        out_shape=jax.ShapeDtypeStruct((1,), jnp.float32),
    )(p, r)
    return (total[0] / _TOTAL).astype(jnp.float32)
